# baseline (speedup 1.0000x reference)
import functools

import jax
import jax.numpy as jnp
from jax import lax
from jax.experimental import pallas as pl
from jax.experimental.pallas import tpu as pltpu
from jax.experimental.pallas import tpu_sc as plsc

_LANES = 16
_CHUNK = 16


@functools.lru_cache(maxsize=None)
def _build(B, S, D, V):
    info = plsc.get_sparse_core_info()
    NC, NS = info.num_cores, info.num_subcores
    NW = NC * NS
    assert S % (NW * _CHUNK) == 0 and D % _LANES == 0
    s_per_w = S // NW
    n_slices = D // _LANES
    n_groups = s_per_w // _CHUNK

    mesh = plsc.VectorSubcoreMesh(core_axis_name="c", subcore_axis_name="s")

    @functools.partial(
        pl.kernel,
        mesh=mesh,
        out_type=jax.ShapeDtypeStruct((B * S, D), jnp.float32),
        scratch_types=(
            [pltpu.VMEM((B, s_per_w), jnp.int32)]
            + [pltpu.VMEM((_CHUNK, D), jnp.float32) for _ in range(2)]
            + [pltpu.VMEM((_CHUNK, D), jnp.float32) for _ in range(2 * B)]
            + [pltpu.SemaphoreType.DMA for _ in range(7)]
        ),
    )
    def emb(x_hbm, tok_hbm, pos_hbm, out_hbm, idx_all, *rest):
        pos_bufs = list(rest[0:2])
        tok_bufs = [list(rest[2 + p * B:2 + (p + 1) * B]) for p in range(2)]
        xsem = rest[2 + 2 * B]
        psems = list(rest[3 + 2 * B:5 + 2 * B])
        gsems = list(rest[5 + 2 * B:7 + 2 * B])
        ssems = list(rest[7 + 2 * B:9 + 2 * B])
        wid = lax.axis_index("s") * NC + lax.axis_index("c")
        sbase = wid * s_per_w

        idx_cps = [
            pltpu.async_copy(x_hbm.at[pl.ds(b * S + sbase, s_per_w)],
                             idx_all.at[b], xsem)
            for b in range(B)
        ]
        for cp in idx_cps:
            cp.wait()

        def pos_load(h):
            p = h % 2
            src = pos_hbm.at[pl.ds(sbase + h * _CHUNK, _CHUNK)]
            return pltpu.async_copy(src, pos_bufs[p], psems[p])

        def gathers(h):
            p = h % 2
            return [
                pltpu.async_copy(
                    tok_hbm.at[idx_all.at[b, pl.ds(h * _CHUNK, _CHUNK)]],
                    tok_bufs[p][b], gsems[p])
                for b in range(B)
            ]

        def stores(h):
            p = h % 2
            return [
                pltpu.async_copy(
                    tok_bufs[p][b],
                    out_hbm.at[pl.ds(b * S + sbase + h * _CHUNK, _CHUNK)],
                    ssems[p])
                for b in range(B)
            ]

        pd = {0: pos_load(0)}
        gd = {0: gathers(0)}
        sd = {}
        for h in range(n_groups):
            p = h % 2
            if h + 1 < n_groups:
                if h >= 1:
                    for cp in sd[h - 1]:
                        cp.wait()
                pd[h + 1] = pos_load(h + 1)
                gd[h + 1] = gathers(h + 1)
            pd[h].wait()
            for cp in gd[h]:
                cp.wait()
            pbuf = pos_bufs[p]
            tbufs = tok_bufs[p]

            @plsc.parallel_loop(0, _CHUNK, step=1, unroll=2)
            def row(r, pbuf=pbuf, tbufs=tbufs):
                for j in range(n_slices):
                    sl = pl.ds(j * _LANES, _LANES)
                    v = pbuf[r, sl]
                    for b in range(B):
                        plsc.addupdate(tbufs[b].at[r, sl], v)

            sd[h] = stores(h)
        for h in (n_groups - 2, n_groups - 1):
            for cp in sd[h]:
                cp.wait()

    return emb


def kernel(x, tok_table, pos_table):
    B, S = x.shape
    V, D = tok_table.shape
    out_flat = _build(B, S, D, V)(x.reshape(B * S), tok_table, pos_table)
    return out_flat.reshape(B, S, D)


# speedup vs baseline: 1.0413x; 1.0413x over previous
import functools

import jax
import jax.numpy as jnp
from jax import lax
from jax.experimental import pallas as pl
from jax.experimental.pallas import tpu as pltpu
from jax.experimental.pallas import tpu_sc as plsc

_LANES = 16
_CHUNK = 8
_NSET = 3


@functools.lru_cache(maxsize=None)
def _build(B, S, D, V):
    info = plsc.get_sparse_core_info()
    NC, NS = info.num_cores, info.num_subcores
    NW = NC * NS
    assert S % (NW * _CHUNK) == 0 and D % _LANES == 0
    s_per_w = S // NW
    n_slices = D // _LANES
    n_groups = s_per_w // _CHUNK

    mesh = plsc.VectorSubcoreMesh(core_axis_name="c", subcore_axis_name="s")

    @functools.partial(
        pl.kernel,
        mesh=mesh,
        out_type=jax.ShapeDtypeStruct((B * S, D), jnp.float32),
        scratch_types=(
            [pltpu.VMEM((B, s_per_w), jnp.int32)]
            + [pltpu.VMEM((_CHUNK, D), jnp.float32) for _ in range(_NSET)]
            + [pltpu.VMEM((_CHUNK, D), jnp.float32) for _ in range(_NSET * B)]
            + [pltpu.SemaphoreType.DMA for _ in range(1 + 3 * _NSET)]
        ),
    )
    def emb(x_hbm, tok_hbm, pos_hbm, out_hbm, idx_all, *rest):
        pos_bufs = list(rest[0:_NSET])
        tok_bufs = [list(rest[_NSET + p * B:_NSET + (p + 1) * B])
                    for p in range(_NSET)]
        nb = _NSET + _NSET * B
        xsem = rest[nb]
        psems = list(rest[nb + 1:nb + 1 + _NSET])
        gsems = list(rest[nb + 1 + _NSET:nb + 1 + 2 * _NSET])
        ssems = list(rest[nb + 1 + 2 * _NSET:nb + 1 + 3 * _NSET])
        wid = lax.axis_index("s") * NC + lax.axis_index("c")
        sbase = wid * s_per_w

        idx_cps = [
            pltpu.async_copy(x_hbm.at[pl.ds(b * S + sbase, s_per_w)],
                             idx_all.at[b], xsem)
            for b in range(B)
        ]
        for cp in idx_cps:
            cp.wait()

        def pos_load(h):
            p = h % _NSET
            src = pos_hbm.at[pl.ds(sbase + h * _CHUNK, _CHUNK)]
            return pltpu.async_copy(src, pos_bufs[p], psems[p])

        def gathers(h):
            p = h % _NSET
            return [
                pltpu.async_copy(
                    tok_hbm.at[idx_all.at[b, pl.ds(h * _CHUNK, _CHUNK)]],
                    tok_bufs[p][b], gsems[p])
                for b in range(B)
            ]

        def stores(h):
            p = h % _NSET
            return [
                pltpu.async_copy(
                    tok_bufs[p][b],
                    out_hbm.at[pl.ds(b * S + sbase + h * _CHUNK, _CHUNK)],
                    ssems[p])
                for b in range(B)
            ]

        pd = {0: pos_load(0)}
        gd = {0: gathers(0)}
        sd = {}
        for h in range(n_groups):
            p = h % _NSET
            if h + 1 < n_groups:
                if h - 2 >= 0:
                    for cp in sd[h - 2]:
                        cp.wait()
                pd[h + 1] = pos_load(h + 1)
                gd[h + 1] = gathers(h + 1)
            pd[h].wait()
            for cp in gd[h]:
                cp.wait()
            pbuf = pos_bufs[p]
            tbufs = tok_bufs[p]

            @plsc.parallel_loop(0, _CHUNK, step=1, unroll=1)
            def row(r, pbuf=pbuf, tbufs=tbufs):
                for j in range(n_slices):
                    sl = pl.ds(j * _LANES, _LANES)
                    v = pbuf[r, sl]
                    for b in range(B):
                        plsc.addupdate(tbufs[b].at[r, sl], v)

            sd[h] = stores(h)
        for h in (n_groups - 2, n_groups - 1):
            for cp in sd[h]:
                cp.wait()

    return emb


def kernel(x, tok_table, pos_table):
    B, S = x.shape
    V, D = tok_table.shape
    out_flat = _build(B, S, D, V)(x.reshape(B * S), tok_table, pos_table)
    return out_flat.reshape(B, S, D)


# speedup vs baseline: 1.0450x; 1.0035x over previous
import functools

import jax
import jax.numpy as jnp
from jax import lax
from jax.experimental import pallas as pl
from jax.experimental.pallas import tpu as pltpu
from jax.experimental.pallas import tpu_sc as plsc

_LANES = 16
_CHUNK = 8
_NSET = 4
_DEPTH = 2


@functools.lru_cache(maxsize=None)
def _build(B, S, D, V):
    info = plsc.get_sparse_core_info()
    NC, NS = info.num_cores, info.num_subcores
    NW = NC * NS
    assert S % (NW * _CHUNK) == 0 and D % _LANES == 0
    s_per_w = S // NW
    n_slices = D // _LANES
    n_groups = s_per_w // _CHUNK

    mesh = plsc.VectorSubcoreMesh(core_axis_name="c", subcore_axis_name="s")

    @functools.partial(
        pl.kernel,
        mesh=mesh,
        out_type=jax.ShapeDtypeStruct((B * S, D), jnp.float32),
        scratch_types=(
            [pltpu.VMEM((B, s_per_w), jnp.int32)]
            + [pltpu.VMEM((_CHUNK, D), jnp.float32) for _ in range(_NSET)]
            + [pltpu.VMEM((_CHUNK, D), jnp.float32) for _ in range(_NSET * B)]
            + [pltpu.SemaphoreType.DMA for _ in range(1 + 3 * _NSET)]
        ),
    )
    def emb(x_hbm, tok_hbm, pos_hbm, out_hbm, idx_all, *rest):
        pos_bufs = list(rest[0:_NSET])
        tok_bufs = [list(rest[_NSET + p * B:_NSET + (p + 1) * B])
                    for p in range(_NSET)]
        nb = _NSET + _NSET * B
        xsem = rest[nb]
        psems = list(rest[nb + 1:nb + 1 + _NSET])
        gsems = list(rest[nb + 1 + _NSET:nb + 1 + 2 * _NSET])
        ssems = list(rest[nb + 1 + 2 * _NSET:nb + 1 + 3 * _NSET])
        wid = lax.axis_index("s") * NC + lax.axis_index("c")
        sbase = wid * s_per_w

        idx_cps = [
            pltpu.async_copy(x_hbm.at[pl.ds(b * S + sbase, s_per_w)],
                             idx_all.at[b], xsem)
            for b in range(B)
        ]
        for cp in idx_cps:
            cp.wait()

        def pos_load(h):
            p = h % _NSET
            src = pos_hbm.at[pl.ds(sbase + h * _CHUNK, _CHUNK)]
            return pltpu.async_copy(src, pos_bufs[p], psems[p])

        def gathers(h):
            p = h % _NSET
            return [
                pltpu.async_copy(
                    tok_hbm.at[idx_all.at[b, pl.ds(h * _CHUNK, _CHUNK)]],
                    tok_bufs[p][b], gsems[p])
                for b in range(B)
            ]

        def stores(h):
            p = h % _NSET
            return [
                pltpu.async_copy(
                    tok_bufs[p][b],
                    out_hbm.at[pl.ds(b * S + sbase + h * _CHUNK, _CHUNK)],
                    ssems[p])
                for b in range(B)
            ]

        pd = {h: pos_load(h) for h in range(_DEPTH)}
        gd = {h: gathers(h) for h in range(_DEPTH)}
        sd = {}
        for h in range(n_groups):
            p = h % _NSET
            g = h + _DEPTH
            if g < n_groups:
                if g - _NSET >= 0:
                    for cp in sd[g - _NSET]:
                        cp.wait()
                pd[g] = pos_load(g)
                gd[g] = gathers(g)
            pd[h].wait()
            for cp in gd[h]:
                cp.wait()
            pbuf = pos_bufs[p]
            tbufs = tok_bufs[p]

            @plsc.parallel_loop(0, _CHUNK, step=1, unroll=1)
            def row(r, pbuf=pbuf, tbufs=tbufs):
                for j in range(n_slices):
                    sl = pl.ds(j * _LANES, _LANES)
                    v = pbuf[r, sl]
                    for b in range(B):
                        plsc.addupdate(tbufs[b].at[r, sl], v)

            sd[h] = stores(h)
        for h in range(max(0, n_groups - _NSET), n_groups):
            for cp in sd[h]:
                cp.wait()

    return emb


def kernel(x, tok_table, pos_table):
    B, S = x.shape
    V, D = tok_table.shape
    out_flat = _build(B, S, D, V)(x.reshape(B * S), tok_table, pos_table)
    return out_flat.reshape(B, S, D)
